# baseline (speedup 1.0000x reference)
import functools

import jax
import jax.numpy as jnp
from jax import lax
from jax.experimental import pallas as pl
from jax.experimental.pallas import tpu as pltpu
from jax.experimental.pallas import tpu_sc as plsc

EMB_DIM = 64
BATCH = 16384
NC = 2
NS = 16
NW = NC * NS
B_PER_W = BATCH // NW
NCHUNK = 4
CHUNK = B_PER_W // NCHUNK
NGROUP = CHUNK // 16


def _sc_body(users_hbm, items_hbm, uf_hbm, vf_hbm, ub_hbm, ib_hbm, out_hbm,
             idxu, idxv, urows, vrows, bu, bi, outv,
             sem_u, sem_v, sem_b):
    wid = lax.axis_index("s") * NC + lax.axis_index("c")

    pltpu.sync_copy(users_hbm.at[wid], idxu)
    pltpu.sync_copy(items_hbm.at[wid], idxv)

    cps = []
    for j in range(NCHUNK):
        cu = pltpu.async_copy(uf_hbm.at[idxu.at[j]], urows.at[j], sem_u)
        cv = pltpu.async_copy(vf_hbm.at[idxv.at[j]], vrows.at[j], sem_v)
        cbu = pltpu.async_copy(ub_hbm.at[idxu.at[j]], bu.at[j], sem_b)
        cbi = pltpu.async_copy(ib_hbm.at[idxv.at[j]], bi.at[j], sem_b)
        cps.append((cu, cv, cbu, cbi))

    lane = lax.iota(jnp.int32, 16)
    for j in range(NCHUNK):
        for c in cps[j]:
            c.wait()
        jj = jnp.full((16,), j, jnp.int32)
        for g in range(NGROUP):
            rows = jnp.full((16,), g * 16, jnp.int32) + lane
            acc0 = bu[j, pl.ds(g * 16, 16)] + bi[j, pl.ds(g * 16, 16)]

            def dbody(d, acc, jj=jj, rows=rows):
                dd = jnp.full((16,), 1, jnp.int32) * d
                du = plsc.load_gather(urows, [jj, rows, dd])
                dv = plsc.load_gather(vrows, [jj, rows, dd])
                return acc + du * dv

            acc = lax.fori_loop(0, EMB_DIM, dbody, acc0, unroll=8)
            outv[j, g] = acc

    pltpu.sync_copy(outv, out_hbm.at[wid])


@jax.jit
def _run(users_r, items_r, uf, vf, ub, ib):
    mesh = plsc.VectorSubcoreMesh(core_axis_name="c", subcore_axis_name="s")
    k = functools.partial(
        pl.kernel,
        mesh=mesh,
        compiler_params=pltpu.CompilerParams(
            needs_layout_passes=False, use_tc_tiling_on_sc=False
        ),
        out_type=jax.ShapeDtypeStruct((NW, NCHUNK, NGROUP, 16), jnp.float32),
        scratch_types=[
            pltpu.VMEM((NCHUNK, CHUNK), jnp.int32),
            pltpu.VMEM((NCHUNK, CHUNK), jnp.int32),
            pltpu.VMEM((NCHUNK, CHUNK, EMB_DIM), jnp.float32),
            pltpu.VMEM((NCHUNK, CHUNK, EMB_DIM), jnp.float32),
            pltpu.VMEM((NCHUNK, CHUNK), jnp.float32),
            pltpu.VMEM((NCHUNK, CHUNK), jnp.float32),
            pltpu.VMEM((NCHUNK, NGROUP, 16), jnp.float32),
            pltpu.SemaphoreType.DMA,
            pltpu.SemaphoreType.DMA,
            pltpu.SemaphoreType.DMA,
        ],
    )(_sc_body)
    return k(users_r, items_r, uf, vf, ub, ib)


def kernel(users, items, user_factors, item_factors, user_biases, item_biases):
    users_r = users.astype(jnp.int32).reshape(NW, NCHUNK, CHUNK)
    items_r = items.astype(jnp.int32).reshape(NW, NCHUNK, CHUNK)
    ub = user_biases.reshape(-1)
    ib = item_biases.reshape(-1)
    out = _run(users_r, items_r, user_factors, item_factors, ub, ib)
    return out.reshape(BATCH)
